# speedup vs baseline: 2.9658x; 2.9658x over previous
import functools

import jax
import jax.numpy as jnp
from jax import lax
from jax.experimental import pallas as pl
from jax.experimental.pallas import tpu as pltpu
from jax.experimental.pallas import tpu_sc as plsc

_BATCH, _HIST, _EMB = 4096, 50, 128
_N = _BATCH * _HIST
_NC, _NS = 2, 16
_NW = _NC * _NS
_RPW = _N // _NW
_CHUNK = 128
_NCH = _RPW // _CHUNK


def _gather_body(idx_hbm, table_hbm, out_hbm, idx_v, buf, gsem):
    wid = lax.axis_index("s") * _NC + lax.axis_index("c")
    pltpu.sync_copy(idx_hbm.at[wid], idx_v)

    def step(j, carry):
        pltpu.async_copy(table_hbm.at[idx_v.at[j]], buf, gsem).wait()
        pltpu.sync_copy(buf, out_hbm.at[wid, j])
        return carry

    lax.fori_loop(0, _NCH, step, 0)


def kernel(x, table):
    idx = x.reshape(_NW, _NCH, _CHUNK).astype(jnp.int32)
    mesh = plsc.VectorSubcoreMesh(core_axis_name="c", subcore_axis_name="s")
    run = functools.partial(
        pl.kernel,
        mesh=mesh,
        out_type=jax.ShapeDtypeStruct((_NW, _NCH, _CHUNK, _EMB), jnp.float32),
        scratch_types=[
            pltpu.VMEM((_NCH, _CHUNK), jnp.int32),
            pltpu.VMEM((_CHUNK, _EMB), jnp.float32),
            pltpu.SemaphoreType.DMA,
        ],
    )(_gather_body)
    out = run(idx, table)
    return out.reshape(_BATCH, _HIST, _EMB)


# speedup vs baseline: 3.2990x; 1.1124x over previous
import functools

import jax
import jax.numpy as jnp
from jax import lax
from jax.experimental import pallas as pl
from jax.experimental.pallas import tpu as pltpu
from jax.experimental.pallas import tpu_sc as plsc

_BATCH, _HIST, _EMB = 4096, 50, 128
_N = _BATCH * _HIST
_NC, _NS = 2, 16
_NW = _NC * _NS
_RPW = _N // _NW
_CHUNK = 80
_NCH = _RPW // _CHUNK
_NBUF = 5
_NGRP = _NCH // _NBUF
_HALF = _NGRP // 2


def _gather_body(idx_hbm, table_hbm, out_hbm, idx_v, bufs,
                 gsem0, gsem1, osem0, osem1):
    wid = lax.axis_index("s") * _NC + lax.axis_index("c")
    pltpu.sync_copy(idx_hbm.at[wid], idx_v)

    def start_gathers(g, s, sem):
        for b in range(_NBUF):
            pltpu.async_copy(table_hbm.at[idx_v.at[g * _NBUF + b]],
                             bufs.at[s * _NBUF + b], sem)

    def wait_gathers(s, sem):
        for b in range(_NBUF):
            pltpu.make_async_copy(table_hbm.at[idx_v.at[b]],
                                  bufs.at[s * _NBUF + b], sem).wait()

    def start_outs(g, s, sem):
        for b in range(_NBUF):
            pltpu.async_copy(bufs.at[s * _NBUF + b],
                             out_hbm.at[wid, g * _NBUF + b], sem)

    def wait_outs(s, sem):
        for b in range(_NBUF):
            pltpu.make_async_copy(bufs.at[s * _NBUF + b],
                                  out_hbm.at[wid, b], sem).wait()

    start_gathers(0, 0, gsem0)

    def pair(h, carry):
        g0 = 2 * h
        wait_gathers(0, gsem0)
        start_gathers(g0 + 1, 1, gsem1)
        start_outs(g0, 0, osem0)
        wait_gathers(1, gsem1)
        wait_outs(0, osem0)
        start_gathers(lax.rem(g0 + 2, _NGRP), 0, gsem0)
        start_outs(g0 + 1, 1, osem1)
        wait_outs(1, osem1)
        return carry

    lax.fori_loop(0, _HALF, pair, 0)
    wait_gathers(0, gsem0)


def kernel(x, table):
    idx = x.reshape(_NW, _NCH, _CHUNK).astype(jnp.int32)
    mesh = plsc.VectorSubcoreMesh(core_axis_name="c", subcore_axis_name="s")
    run = functools.partial(
        pl.kernel,
        mesh=mesh,
        out_type=jax.ShapeDtypeStruct((_NW, _NCH, _CHUNK, _EMB), jnp.float32),
        scratch_types=[
            pltpu.VMEM((_NCH, _CHUNK), jnp.int32),
            pltpu.VMEM((2 * _NBUF, _CHUNK, _EMB), jnp.float32),
            pltpu.SemaphoreType.DMA,
            pltpu.SemaphoreType.DMA,
            pltpu.SemaphoreType.DMA,
            pltpu.SemaphoreType.DMA,
        ],
    )(_gather_body)
    out = run(idx, table)
    return out.reshape(_BATCH, _HIST, _EMB)


# speedup vs baseline: 5.8035x; 1.7591x over previous
import functools

import jax
import jax.numpy as jnp
from jax import lax
from jax.experimental import pallas as pl
from jax.experimental.pallas import tpu as pltpu
from jax.experimental.pallas import tpu_sc as plsc

_BATCH, _HIST, _EMB = 4096, 50, 128
_N = _BATCH * _HIST
_NC, _NS = 2, 16
_NW = _NC * _NS
_RPW = _N // _NW
_CHUNK = _HIST
_NCH = _RPW // _CHUNK
_NBUF = 8
_NGRP = _NCH // _NBUF
_HALF = _NGRP // 2


def _gather_body(idx_hbm, table_hbm, out_hbm, idx_v, bufs,
                 gsem0, gsem1, osem0, osem1):
    wid = lax.axis_index("s") * _NC + lax.axis_index("c")
    row0 = wid * _NCH
    pltpu.sync_copy(idx_hbm.at[wid], idx_v)

    def start_gathers(g, s, sem):
        for b in range(_NBUF):
            pltpu.async_copy(table_hbm.at[idx_v.at[g * _NBUF + b]],
                             bufs.at[s * _NBUF + b], sem)

    def wait_gathers(s, sem):
        for b in range(_NBUF):
            pltpu.make_async_copy(table_hbm.at[idx_v.at[b]],
                                  bufs.at[s * _NBUF + b], sem).wait()

    def start_outs(g, s, sem):
        for b in range(_NBUF):
            pltpu.async_copy(bufs.at[s * _NBUF + b],
                             out_hbm.at[row0 + g * _NBUF + b], sem)

    def wait_outs(s, sem):
        for b in range(_NBUF):
            pltpu.make_async_copy(bufs.at[s * _NBUF + b],
                                  out_hbm.at[row0 + b], sem).wait()

    start_gathers(0, 0, gsem0)

    def pair(h, carry):
        g0 = 2 * h
        wait_gathers(0, gsem0)
        start_gathers(g0 + 1, 1, gsem1)
        start_outs(g0, 0, osem0)
        wait_gathers(1, gsem1)
        wait_outs(0, osem0)
        start_gathers(lax.rem(g0 + 2, _NGRP), 0, gsem0)
        start_outs(g0 + 1, 1, osem1)
        wait_outs(1, osem1)
        return carry

    lax.fori_loop(0, _HALF, pair, 0)
    wait_gathers(0, gsem0)


def kernel(x, table):
    idx = x.reshape(_NW, _NCH, _CHUNK).astype(jnp.int32)
    mesh = plsc.VectorSubcoreMesh(core_axis_name="c", subcore_axis_name="s")
    run = functools.partial(
        pl.kernel,
        mesh=mesh,
        out_type=jax.ShapeDtypeStruct((_BATCH, _HIST, _EMB), jnp.float32),
        scratch_types=[
            pltpu.VMEM((_NCH, _CHUNK), jnp.int32),
            pltpu.VMEM((2 * _NBUF, _CHUNK, _EMB), jnp.float32),
            pltpu.SemaphoreType.DMA,
            pltpu.SemaphoreType.DMA,
            pltpu.SemaphoreType.DMA,
            pltpu.SemaphoreType.DMA,
        ],
    )(_gather_body)
    return run(idx, table)
